# baseline (speedup 1.0000x reference)
import functools

import jax
import jax.numpy as jnp
from jax import lax
from jax.experimental import pallas as pl
from jax.experimental.pallas import tpu as pltpu
from jax.experimental.pallas import tpu_sc as plsc

K = 16
QT = 256
ROWW = 128
OUTW = 3 + 64


G = 128


def _topk_body(xyz_ref, sxyz_ref, idx_ref):
    q = xyz_ref[0]
    s = sxyz_ref[0]
    n = s.shape[0]
    gs = n // G
    d = -2.0 * lax.dot_general(s, q, (((1,), (1,)), ((), ())),
                               preferred_element_type=jnp.float32)
    q2 = jnp.sum(q * q, axis=1)
    s2 = jnp.sum(s * s, axis=1)
    d = d + q2[None, :]
    d = d + s2[:, None]
    qt = d.shape[1]
    d3 = d.reshape(G, gs, qt)
    liota = lax.broadcasted_iota(jnp.int32, (G, gs, qt), 1).astype(jnp.float32)
    gbase = lax.broadcasted_iota(jnp.int32, (G, qt), 0) * gs
    inf = jnp.float32(jnp.inf)
    gsf = jnp.float32(gs)

    def merge(W, WI, cv, ci):
        ev = jnp.concatenate([W, cv], axis=0)
        ei = jnp.concatenate([WI, ci], axis=0)
        nW, nWI = [], []
        for _ in range(K):
            w = jnp.min(ev, axis=0)
            wm = ev == w[None, :]
            wi = jnp.min(jnp.where(wm, ei, n), axis=0)
            nW.append(w)
            nWI.append(wi)
            ev = jnp.where(wm & (ei == wi[None, :]), inf, ev)
        return jnp.stack(nW), jnp.stack(nWI)

    def cond(st):
        r, done = st[0], st[1]
        return jnp.logical_and(r < K + 1, jnp.logical_not(done))

    def body(st):
        r, _, d3, W, WI = st
        gmin = jnp.min(d3, axis=1)
        gamf = jnp.min(jnp.where(d3 == gmin[:, None, :], liota, gsf), axis=1)
        gam = gamf.astype(jnp.int32) + gbase
        t, ti = W[K - 1], WI[K - 1]
        bad = (gmin < t[None, :]) | ((gmin == t[None, :]) & (gam < ti[None, :]))
        W, WI = merge(W, WI, gmin, gam)
        d3 = jnp.where(liota == gamf[:, None, :], inf, d3)
        return r + 1, jnp.logical_not(jnp.any(bad)), d3, W, WI

    st = (jnp.int32(0), jnp.bool_(False), d3,
          jnp.full((K, qt), inf), jnp.full((K, qt), n, jnp.int32))
    _, _, _, W, WI = lax.while_loop(cond, body, st)
    idx_ref[0] = WI


def _topk(s_xyz, xyz):
    B, N, _ = s_xyz.shape
    S = xyz.shape[1]
    return pl.pallas_call(
        _topk_body,
        grid=(B, S // QT),
        in_specs=[
            pl.BlockSpec((1, QT, 3), lambda b, i: (b, i, 0)),
            pl.BlockSpec((1, N, 3), lambda b, i: (b, 0, 0)),
        ],
        out_specs=pl.BlockSpec((1, K, QT), lambda b, i: (b, 0, i)),
        out_shape=jax.ShapeDtypeStruct((B, K, S), jnp.int32),
    )(xyz, s_xyz)


def _make_sc_gather(BS):
    NC, NS = 2, 16
    NW = NC * NS
    QW = BS // NW
    NQ = 8
    NB = QW // NQ
    mesh = plsc.VectorSubcoreMesh(core_axis_name="c", subcore_axis_name="s")

    @functools.partial(
        pl.kernel, mesh=mesh,
        out_type=[
            jax.ShapeDtypeStruct((BS * K * OUTW,), jnp.float32),
            jax.ShapeDtypeStruct((BS * K * 3,), jnp.float32),
        ],
        scratch_types=[
            pltpu.VMEM((NQ * K,), jnp.int32),
            pltpu.VMEM((NQ * K, ROWW), jnp.float32),
            pltpu.VMEM((NQ, 16), jnp.float32),
            pltpu.VMEM((NQ * K * OUTW + 16,), jnp.float32),
            pltpu.VMEM((NQ * K * 3 + 16,), jnp.float32),
            pltpu.SemaphoreType.DMA,
        ],
    )
    def sc_gather(feat_hbm, gidx_hbm, qpad_hbm, newp_hbm, gxyz_hbm,
                  idx_v, rows_v, q_v, out_v, gx_v, sem):
        wid = lax.axis_index("s") * NC + lax.axis_index("c")

        def block(t, _):
            qbase = wid * QW + t * NQ
            pltpu.sync_copy(gidx_hbm.at[pl.ds(qbase * K, NQ * K)], idx_v)
            pltpu.async_copy(feat_hbm.at[idx_v], rows_v, sem).wait()
            pltpu.sync_copy(qpad_hbm.at[pl.ds(qbase, NQ)], q_v)

            def body(i, _):
                qvec = q_v[i, :]
                for r in range(K):
                    row = i * K + r
                    d0 = row * OUTW
                    v0 = rows_v[row, pl.ds(0, 16)] - qvec
                    out_v[pl.ds(d0, 16)] = v0
                    for j in range(1, 5):
                        out_v[pl.ds(d0 + 16 * j, 16)] = rows_v[row, pl.ds(16 * j, 16)]
                    gx_v[pl.ds(row * 3, 16)] = v0
                return 0

            lax.fori_loop(0, NQ, body, 0)
            pltpu.sync_copy(out_v.at[pl.ds(0, NQ * K * OUTW)],
                            newp_hbm.at[pl.ds(qbase * K * OUTW, NQ * K * OUTW)])
            pltpu.sync_copy(gx_v.at[pl.ds(0, NQ * K * 3)],
                            gxyz_hbm.at[pl.ds(qbase * K * 3, NQ * K * 3)])
            return 0

        lax.fori_loop(0, NB, block, 0)

    return sc_gather


def kernel(s_xyz, xyz, s_points, nsample):
    B, N, _ = s_xyz.shape
    S = xyz.shape[1]
    D = s_points.shape[2]
    BS = B * S

    idx = _topk(s_xyz, xyz)
    idx = jnp.transpose(idx, (0, 2, 1))

    pad = jnp.zeros((B, N, ROWW - 3 - D), jnp.float32)
    feat = jnp.concatenate([s_xyz, s_points, pad], axis=-1).reshape(B * N, ROWW)
    gidx = (idx + (jnp.arange(B, dtype=jnp.int32) * N)[:, None, None]
            ).reshape(BS * K)
    qpad = jnp.concatenate(
        [xyz, jnp.zeros((B, S, 13), jnp.float32)], axis=-1).reshape(BS, 16)

    newp_flat, gxyz_flat = _make_sc_gather(BS)(feat, gidx, qpad)
    new_points = newp_flat.reshape(B, S, K, OUTW)
    grouped_xyz_norm = gxyz_flat.reshape(B, S, K, 3)
    return new_points, grouped_xyz_norm


# speedup vs baseline: 1.0001x; 1.0001x over previous
import functools

import jax
import jax.numpy as jnp
from jax import lax
from jax.experimental import pallas as pl
from jax.experimental.pallas import tpu as pltpu
from jax.experimental.pallas import tpu_sc as plsc

K = 16
QT = 256
ROWW = 128
OUTW = 3 + 64


G = 128


def _topk_body(xyz_ref, sxyz_ref, idx_ref):
    q = xyz_ref[0]
    s = sxyz_ref[0]
    n = s.shape[0]
    gs = n // G
    d = -2.0 * lax.dot_general(s, q, (((1,), (1,)), ((), ())),
                               preferred_element_type=jnp.float32)
    q2 = jnp.sum(q * q, axis=1)
    s2 = jnp.sum(s * s, axis=1)
    d = d + q2[None, :]
    d = d + s2[:, None]
    qt = d.shape[1]
    d3 = d.reshape(G, gs, qt)
    liota = lax.broadcasted_iota(jnp.int32, (G, gs, qt), 1).astype(jnp.float32)
    gbase = lax.broadcasted_iota(jnp.int32, (G, qt), 0) * gs
    inf = jnp.float32(jnp.inf)
    gsf = jnp.float32(gs)

    def merge(W, WI, cv, ci):
        ev = jnp.concatenate([W, cv], axis=0)
        ei = jnp.concatenate([WI, ci], axis=0)
        nW, nWI = [], []
        for _ in range(K):
            w = jnp.min(ev, axis=0)
            wm = ev == w[None, :]
            wi = jnp.min(jnp.where(wm, ei, n), axis=0)
            nW.append(w)
            nWI.append(wi)
            ev = jnp.where(wm & (ei == wi[None, :]), inf, ev)
        return jnp.stack(nW), jnp.stack(nWI)

    def cond(st):
        r, done = st[0], st[1]
        return jnp.logical_and(r < K + 1, jnp.logical_not(done))

    def body(st):
        r, _, d3, W, WI, lio = st
        gmin = jnp.min(d3, axis=1)
        gamf = jnp.min(jnp.where(d3 == gmin[:, None, :], lio, gsf), axis=1)
        gam = gamf.astype(jnp.int32) + gbase
        t, ti = W[K - 1], WI[K - 1]
        bad = (gmin < t[None, :]) | ((gmin == t[None, :]) & (gam < ti[None, :]))
        W, WI = merge(W, WI, gmin, gam)
        d3 = jnp.where(lio == gamf[:, None, :], inf, d3)
        return r + 1, jnp.logical_not(jnp.any(bad)), d3, W, WI, lio

    st = (jnp.int32(0), jnp.bool_(False), d3,
          jnp.full((K, qt), inf), jnp.full((K, qt), n, jnp.int32), liota)
    _, _, _, W, WI, _ = lax.while_loop(cond, body, st)
    idx_ref[0] = WI


def _topk(s_xyz, xyz):
    B, N, _ = s_xyz.shape
    S = xyz.shape[1]
    return pl.pallas_call(
        _topk_body,
        grid=(B, S // QT),
        in_specs=[
            pl.BlockSpec((1, QT, 3), lambda b, i: (b, i, 0)),
            pl.BlockSpec((1, N, 3), lambda b, i: (b, 0, 0)),
        ],
        out_specs=pl.BlockSpec((1, K, QT), lambda b, i: (b, 0, i)),
        out_shape=jax.ShapeDtypeStruct((B, K, S), jnp.int32),
    )(xyz, s_xyz)


def _make_sc_gather(BS):
    NC, NS = 2, 16
    NW = NC * NS
    QW = BS // NW
    NQ = 8
    NB = QW // NQ
    mesh = plsc.VectorSubcoreMesh(core_axis_name="c", subcore_axis_name="s")

    @functools.partial(
        pl.kernel, mesh=mesh,
        out_type=[
            jax.ShapeDtypeStruct((BS * K * OUTW,), jnp.float32),
            jax.ShapeDtypeStruct((BS * K * 3,), jnp.float32),
        ],
        scratch_types=[
            pltpu.VMEM((NQ * K,), jnp.int32),
            pltpu.VMEM((NQ * K, ROWW), jnp.float32),
            pltpu.VMEM((NQ, 16), jnp.float32),
            pltpu.VMEM((NQ * K * OUTW + 16,), jnp.float32),
            pltpu.VMEM((NQ * K * 3 + 16,), jnp.float32),
            pltpu.SemaphoreType.DMA,
        ],
    )
    def sc_gather(feat_hbm, gidx_hbm, qpad_hbm, newp_hbm, gxyz_hbm,
                  idx_v, rows_v, q_v, out_v, gx_v, sem):
        wid = lax.axis_index("s") * NC + lax.axis_index("c")

        def block(t, _):
            qbase = wid * QW + t * NQ
            pltpu.sync_copy(gidx_hbm.at[pl.ds(qbase * K, NQ * K)], idx_v)
            pltpu.async_copy(feat_hbm.at[idx_v], rows_v, sem).wait()
            pltpu.sync_copy(qpad_hbm.at[pl.ds(qbase, NQ)], q_v)

            def body(i, _):
                qvec = q_v[i, :]
                for r in range(K):
                    row = i * K + r
                    d0 = row * OUTW
                    v0 = rows_v[row, pl.ds(0, 16)] - qvec
                    out_v[pl.ds(d0, 16)] = v0
                    for j in range(1, 5):
                        out_v[pl.ds(d0 + 16 * j, 16)] = rows_v[row, pl.ds(16 * j, 16)]
                    gx_v[pl.ds(row * 3, 16)] = v0
                return 0

            lax.fori_loop(0, NQ, body, 0)
            pltpu.sync_copy(out_v.at[pl.ds(0, NQ * K * OUTW)],
                            newp_hbm.at[pl.ds(qbase * K * OUTW, NQ * K * OUTW)])
            pltpu.sync_copy(gx_v.at[pl.ds(0, NQ * K * 3)],
                            gxyz_hbm.at[pl.ds(qbase * K * 3, NQ * K * 3)])
            return 0

        lax.fori_loop(0, NB, block, 0)

    return sc_gather


def kernel(s_xyz, xyz, s_points, nsample):
    B, N, _ = s_xyz.shape
    S = xyz.shape[1]
    D = s_points.shape[2]
    BS = B * S

    idx = _topk(s_xyz, xyz)
    idx = jnp.transpose(idx, (0, 2, 1))

    pad = jnp.zeros((B, N, ROWW - 3 - D), jnp.float32)
    feat = jnp.concatenate([s_xyz, s_points, pad], axis=-1).reshape(B * N, ROWW)
    gidx = (idx + (jnp.arange(B, dtype=jnp.int32) * N)[:, None, None]
            ).reshape(BS * K)
    qpad = jnp.concatenate(
        [xyz, jnp.zeros((B, S, 13), jnp.float32)], axis=-1).reshape(BS, 16)

    newp_flat, gxyz_flat = _make_sc_gather(BS)(feat, gidx, qpad)
    new_points = newp_flat.reshape(B, S, K, OUTW)
    grouped_xyz_norm = gxyz_flat.reshape(B, S, K, 3)
    return new_points, grouped_xyz_norm


# speedup vs baseline: 1.3048x; 1.3047x over previous
import functools

import jax
import jax.numpy as jnp
from jax import lax
from jax.experimental import pallas as pl
from jax.experimental.pallas import tpu as pltpu
from jax.experimental.pallas import tpu_sc as plsc

K = 16
QT = 256
ROWW = 128
OUTW = 3 + 64


G = 128


def _topk_body(xyz_ref, sxyz_ref, idx_ref):
    q = xyz_ref[0]
    s = sxyz_ref[0]
    n = s.shape[0]
    gs = n // G
    d = -2.0 * lax.dot_general(s, q, (((1,), (1,)), ((), ())),
                               preferred_element_type=jnp.float32)
    q2 = jnp.sum(q * q, axis=1)
    s2 = jnp.sum(s * s, axis=1)
    d = d + q2[None, :]
    d = d + s2[:, None]
    qt = d.shape[1]
    d3 = d.reshape(G, gs, qt)
    gidx = (lax.broadcasted_iota(jnp.int32, (G, gs, qt), 0) * gs
            + lax.broadcasted_iota(jnp.int32, (G, gs, qt), 1))
    inf = jnp.float32(jnp.inf)

    def cands(d3):
        gmin = jnp.min(d3, axis=1)
        gam = jnp.min(jnp.where(d3 == gmin[:, None, :], gidx, n), axis=1)
        return gmin, gam

    def mask(d3, gam):
        return jnp.where(gidx == gam[:, None, :], inf, d3)

    def merge(W, WI, cv, ci):
        ev = jnp.concatenate([W, cv], axis=0)
        ei = jnp.concatenate([WI, ci], axis=0)
        nW, nWI = [], []
        for _ in range(K):
            w = jnp.min(ev, axis=0)
            wm = ev == w[None, :]
            wi = jnp.min(jnp.where(wm, ei, n), axis=0)
            nW.append(w)
            nWI.append(wi)
            ev = jnp.where(wm & (ei == wi[None, :]), inf, ev)
        return jnp.stack(nW), jnp.stack(nWI)

    gmin, gam = cands(d3)
    W, WI = merge(jnp.full((K, qt), inf), jnp.full((K, qt), n, jnp.int32),
                  gmin, gam)
    for _ in range(2):
        d3 = mask(d3, gam)
        gmin, gam = cands(d3)
        W, WI = merge(W, WI, gmin, gam)

    def check(W, WI, gmin, gam):
        t, ti = W[K - 1], WI[K - 1]
        bad = (gmin < t[None, :]) | ((gmin == t[None, :]) & (gam < ti[None, :]))
        return jnp.logical_not(jnp.any(bad))

    def cond(st):
        r, done = st[0], st[1]
        return jnp.logical_and(r < K, jnp.logical_not(done))

    def body(st):
        r, _, d3, gmin, gam, W, WI = st
        W, WI = merge(W, WI, gmin, gam)
        d3 = mask(d3, gam)
        gmin, gam = cands(d3)
        done = check(W, WI, gmin, gam)
        return r + 1, done, d3, gmin, gam, W, WI

    d3 = mask(d3, gam)
    gmin, gam = cands(d3)
    st = (jnp.int32(3), check(W, WI, gmin, gam), d3, gmin, gam, W, WI)
    _, _, _, _, _, W, WI = lax.while_loop(cond, body, st)
    idx_ref[0] = WI


def _topk(s_xyz, xyz):
    B, N, _ = s_xyz.shape
    S = xyz.shape[1]
    return pl.pallas_call(
        _topk_body,
        grid=(B, S // QT),
        in_specs=[
            pl.BlockSpec((1, QT, 3), lambda b, i: (b, i, 0)),
            pl.BlockSpec((1, N, 3), lambda b, i: (b, 0, 0)),
        ],
        out_specs=pl.BlockSpec((1, K, QT), lambda b, i: (b, 0, i)),
        out_shape=jax.ShapeDtypeStruct((B, K, S), jnp.int32),
    )(xyz, s_xyz)


def _make_sc_gather(BS):
    NC, NS = 2, 16
    NW = NC * NS
    QW = BS // NW
    NQ = 8
    NB = QW // NQ
    mesh = plsc.VectorSubcoreMesh(core_axis_name="c", subcore_axis_name="s")

    @functools.partial(
        pl.kernel, mesh=mesh,
        out_type=[
            jax.ShapeDtypeStruct((BS * K * OUTW,), jnp.float32),
            jax.ShapeDtypeStruct((BS * K * 3,), jnp.float32),
        ],
        scratch_types=[
            pltpu.VMEM((NQ * K,), jnp.int32),
            pltpu.VMEM((NQ * K, ROWW), jnp.float32),
            pltpu.VMEM((NQ, 16), jnp.float32),
            pltpu.VMEM((NQ * K * OUTW + 16,), jnp.float32),
            pltpu.VMEM((NQ * K * 3 + 16,), jnp.float32),
            pltpu.SemaphoreType.DMA,
        ],
    )
    def sc_gather(feat_hbm, gidx_hbm, qpad_hbm, newp_hbm, gxyz_hbm,
                  idx_v, rows_v, q_v, out_v, gx_v, sem):
        wid = lax.axis_index("s") * NC + lax.axis_index("c")

        def block(t, _):
            qbase = wid * QW + t * NQ
            pltpu.sync_copy(gidx_hbm.at[pl.ds(qbase * K, NQ * K)], idx_v)
            pltpu.async_copy(feat_hbm.at[idx_v], rows_v, sem).wait()
            pltpu.sync_copy(qpad_hbm.at[pl.ds(qbase, NQ)], q_v)

            def body(i, _):
                qvec = q_v[i, :]
                for r in range(K):
                    row = i * K + r
                    d0 = row * OUTW
                    v0 = rows_v[row, pl.ds(0, 16)] - qvec
                    out_v[pl.ds(d0, 16)] = v0
                    for j in range(1, 5):
                        out_v[pl.ds(d0 + 16 * j, 16)] = rows_v[row, pl.ds(16 * j, 16)]
                    gx_v[pl.ds(row * 3, 16)] = v0
                return 0

            lax.fori_loop(0, NQ, body, 0)
            pltpu.sync_copy(out_v.at[pl.ds(0, NQ * K * OUTW)],
                            newp_hbm.at[pl.ds(qbase * K * OUTW, NQ * K * OUTW)])
            pltpu.sync_copy(gx_v.at[pl.ds(0, NQ * K * 3)],
                            gxyz_hbm.at[pl.ds(qbase * K * 3, NQ * K * 3)])
            return 0

        lax.fori_loop(0, NB, block, 0)

    return sc_gather


def kernel(s_xyz, xyz, s_points, nsample):
    B, N, _ = s_xyz.shape
    S = xyz.shape[1]
    D = s_points.shape[2]
    BS = B * S

    idx = _topk(s_xyz, xyz)
    idx = jnp.transpose(idx, (0, 2, 1))

    pad = jnp.zeros((B, N, ROWW - 3 - D), jnp.float32)
    feat = jnp.concatenate([s_xyz, s_points, pad], axis=-1).reshape(B * N, ROWW)
    gidx = (idx + (jnp.arange(B, dtype=jnp.int32) * N)[:, None, None]
            ).reshape(BS * K)
    qpad = jnp.concatenate(
        [xyz, jnp.zeros((B, S, 13), jnp.float32)], axis=-1).reshape(BS, 16)

    newp_flat, gxyz_flat = _make_sc_gather(BS)(feat, gidx, qpad)
    new_points = newp_flat.reshape(B, S, K, OUTW)
    grouped_xyz_norm = gxyz_flat.reshape(B, S, K, 3)
    return new_points, grouped_xyz_norm


# speedup vs baseline: 1.3914x; 1.0664x over previous
import functools

import jax
import jax.numpy as jnp
from jax import lax
from jax.experimental import pallas as pl
from jax.experimental.pallas import tpu as pltpu
from jax.experimental.pallas import tpu_sc as plsc

K = 16
QT = 256
ROWW = 128
OUTW = 3 + 64


G = 128


def _topk_body(xyz_ref, sxyz_ref, idx_ref):
    q = xyz_ref[0]
    s = sxyz_ref[0]
    n = s.shape[0]
    gs = n // G
    d = -2.0 * lax.dot_general(s, q, (((1,), (1,)), ((), ())),
                               preferred_element_type=jnp.float32)
    q2 = jnp.sum(q * q, axis=1)
    s2 = jnp.sum(s * s, axis=1)
    d = d + q2[None, :]
    d = d + s2[:, None]
    qt = d.shape[1]
    d3 = d.reshape(G, gs, qt)
    gidx = (lax.broadcasted_iota(jnp.int32, (G, gs, qt), 0) * gs
            + lax.broadcasted_iota(jnp.int32, (G, gs, qt), 1))
    gbase = lax.broadcasted_iota(jnp.int32, (G, qt), 0) * gs
    inf = jnp.float32(jnp.inf)

    def cands(d3):
        v = d3
        i = None
        m = gs
        while m > 8:
            h = m // 2
            va, vb = v[:, :h], v[:, h:]
            take = vb < va
            if i is None:
                ia = lax.broadcasted_iota(jnp.int32, va.shape, 1)
                ib = ia + h
            else:
                ia, ib = i[:, :h], i[:, h:]
            v = jnp.where(take, vb, va)
            i = jnp.where(take, ib, ia)
            m = h
        gmin = jnp.min(v, axis=1)
        gam = jnp.min(jnp.where(v == gmin[:, None, :], i, n), axis=1)
        return gmin, gam + gbase

    def mask(d3, gam):
        return jnp.where(gidx == gam[:, None, :], inf, d3)

    def merge(W, WI, cv, ci):
        ev = jnp.concatenate([W, cv], axis=0)
        ei = jnp.concatenate([WI, ci], axis=0)
        nW, nWI = [], []
        for _ in range(K):
            w = jnp.min(ev, axis=0)
            wm = ev == w[None, :]
            wi = jnp.min(jnp.where(wm, ei, n), axis=0)
            nW.append(w)
            nWI.append(wi)
            ev = jnp.where(wm & (ei == wi[None, :]), inf, ev)
        return jnp.stack(nW), jnp.stack(nWI)

    gmin, gam = cands(d3)
    W, WI = merge(jnp.full((K, qt), inf), jnp.full((K, qt), n, jnp.int32),
                  gmin, gam)
    for _ in range(2):
        d3 = mask(d3, gam)
        gmin, gam = cands(d3)
        W, WI = merge(W, WI, gmin, gam)

    def check(W, WI, gmin, gam):
        t, ti = W[K - 1], WI[K - 1]
        bad = (gmin < t[None, :]) | ((gmin == t[None, :]) & (gam < ti[None, :]))
        return jnp.logical_not(jnp.any(bad))

    def cond(st):
        r, done = st[0], st[1]
        return jnp.logical_and(r < K, jnp.logical_not(done))

    def body(st):
        r, _, d3, gmin, gam, W, WI = st
        W, WI = merge(W, WI, gmin, gam)
        d3 = mask(d3, gam)
        gmin, gam = cands(d3)
        done = check(W, WI, gmin, gam)
        return r + 1, done, d3, gmin, gam, W, WI

    d3 = mask(d3, gam)
    gmin, gam = cands(d3)
    st = (jnp.int32(3), check(W, WI, gmin, gam), d3, gmin, gam, W, WI)
    _, _, _, _, _, W, WI = lax.while_loop(cond, body, st)
    idx_ref[0] = WI


def _topk(s_xyz, xyz):
    B, N, _ = s_xyz.shape
    S = xyz.shape[1]
    return pl.pallas_call(
        _topk_body,
        grid=(B, S // QT),
        in_specs=[
            pl.BlockSpec((1, QT, 3), lambda b, i: (b, i, 0)),
            pl.BlockSpec((1, N, 3), lambda b, i: (b, 0, 0)),
        ],
        out_specs=pl.BlockSpec((1, K, QT), lambda b, i: (b, 0, i)),
        out_shape=jax.ShapeDtypeStruct((B, K, S), jnp.int32),
    )(xyz, s_xyz)


def _make_sc_gather(BS):
    NC, NS = 2, 16
    NW = NC * NS
    QW = BS // NW
    NQ = 8
    NB = QW // NQ
    mesh = plsc.VectorSubcoreMesh(core_axis_name="c", subcore_axis_name="s")

    @functools.partial(
        pl.kernel, mesh=mesh,
        out_type=[
            jax.ShapeDtypeStruct((BS * K * OUTW,), jnp.float32),
            jax.ShapeDtypeStruct((BS * K * 3,), jnp.float32),
        ],
        scratch_types=[
            pltpu.VMEM((NQ * K,), jnp.int32),
            pltpu.VMEM((NQ * K, ROWW), jnp.float32),
            pltpu.VMEM((NQ, 16), jnp.float32),
            pltpu.VMEM((NQ * K * OUTW + 16,), jnp.float32),
            pltpu.VMEM((NQ * K * 3 + 16,), jnp.float32),
            pltpu.SemaphoreType.DMA,
        ],
    )
    def sc_gather(feat_hbm, gidx_hbm, qpad_hbm, newp_hbm, gxyz_hbm,
                  idx_v, rows_v, q_v, out_v, gx_v, sem):
        wid = lax.axis_index("s") * NC + lax.axis_index("c")

        def block(t, _):
            qbase = wid * QW + t * NQ
            pltpu.sync_copy(gidx_hbm.at[pl.ds(qbase * K, NQ * K)], idx_v)
            pltpu.async_copy(feat_hbm.at[idx_v], rows_v, sem).wait()
            pltpu.sync_copy(qpad_hbm.at[pl.ds(qbase, NQ)], q_v)

            def body(i, _):
                qvec = q_v[i, :]
                for r in range(K):
                    row = i * K + r
                    d0 = row * OUTW
                    v0 = rows_v[row, pl.ds(0, 16)] - qvec
                    out_v[pl.ds(d0, 16)] = v0
                    for j in range(1, 5):
                        out_v[pl.ds(d0 + 16 * j, 16)] = rows_v[row, pl.ds(16 * j, 16)]
                    gx_v[pl.ds(row * 3, 16)] = v0
                return 0

            lax.fori_loop(0, NQ, body, 0)
            pltpu.sync_copy(out_v.at[pl.ds(0, NQ * K * OUTW)],
                            newp_hbm.at[pl.ds(qbase * K * OUTW, NQ * K * OUTW)])
            pltpu.sync_copy(gx_v.at[pl.ds(0, NQ * K * 3)],
                            gxyz_hbm.at[pl.ds(qbase * K * 3, NQ * K * 3)])
            return 0

        lax.fori_loop(0, NB, block, 0)

    return sc_gather


def kernel(s_xyz, xyz, s_points, nsample):
    B, N, _ = s_xyz.shape
    S = xyz.shape[1]
    D = s_points.shape[2]
    BS = B * S

    idx = _topk(s_xyz, xyz)
    idx = jnp.transpose(idx, (0, 2, 1))

    pad = jnp.zeros((B, N, ROWW - 3 - D), jnp.float32)
    feat = jnp.concatenate([s_xyz, s_points, pad], axis=-1).reshape(B * N, ROWW)
    gidx = (idx + (jnp.arange(B, dtype=jnp.int32) * N)[:, None, None]
            ).reshape(BS * K)
    qpad = jnp.concatenate(
        [xyz, jnp.zeros((B, S, 13), jnp.float32)], axis=-1).reshape(BS, 16)

    newp_flat, gxyz_flat = _make_sc_gather(BS)(feat, gidx, qpad)
    new_points = newp_flat.reshape(B, S, K, OUTW)
    grouped_xyz_norm = gxyz_flat.reshape(B, S, K, 3)
    return new_points, grouped_xyz_norm


# speedup vs baseline: 1.4566x; 1.0468x over previous
import functools

import jax
import jax.numpy as jnp
from jax import lax
from jax.experimental import pallas as pl
from jax.experimental.pallas import tpu as pltpu
from jax.experimental.pallas import tpu_sc as plsc

K = 16
QT = 256
ROWW = 128
OUTW = 3 + 64


G = 128


def _topk_body(xyz_ref, sxyz_ref, idx_ref):
    q = xyz_ref[0]
    s = sxyz_ref[0]
    n = s.shape[0]
    gs = n // G
    d = -2.0 * lax.dot_general(s, q, (((1,), (1,)), ((), ())),
                               preferred_element_type=jnp.float32)
    q2 = jnp.sum(q * q, axis=1)
    s2 = jnp.sum(s * s, axis=1)
    d = d + q2[None, :]
    d = d + s2[:, None]
    qt = d.shape[1]
    d3 = d.reshape(G, gs, qt)
    gidx = (lax.broadcasted_iota(jnp.int32, (G, gs, qt), 0) * gs
            + lax.broadcasted_iota(jnp.int32, (G, gs, qt), 1))
    gbase = lax.broadcasted_iota(jnp.int32, (G, qt), 0) * gs
    inf = jnp.float32(jnp.inf)

    def cands(d3):
        v = d3
        i = None
        m = gs
        while m > 8:
            h = m // 2
            va, vb = v[:, :h], v[:, h:]
            take = vb < va
            if i is None:
                ia = lax.broadcasted_iota(jnp.int32, va.shape, 1)
                ib = ia + h
            else:
                ia, ib = i[:, :h], i[:, h:]
            v = jnp.where(take, vb, va)
            i = jnp.where(take, ib, ia)
            m = h
        gmin = jnp.min(v, axis=1)
        gam = jnp.min(jnp.where(v == gmin[:, None, :], i, n), axis=1)
        return gmin, gam + gbase

    def mask(d3, gam):
        return jnp.where(gidx == gam[:, None, :], inf, d3)

    def merge(W, WI, cv, ci):
        ev = jnp.concatenate([W, cv], axis=0)
        ei = jnp.concatenate([WI, ci], axis=0)
        nW, nWI = [], []
        for _ in range(K):
            w = jnp.min(ev, axis=0)
            wm = ev == w[None, :]
            wi = jnp.min(jnp.where(wm, ei, n), axis=0)
            nW.append(w)
            nWI.append(wi)
            ev = jnp.where(wm & (ei == wi[None, :]), inf, ev)
        return jnp.stack(nW), jnp.stack(nWI)

    gmin, gam = cands(d3)
    W, WI = merge(jnp.full((K, qt), inf), jnp.full((K, qt), n, jnp.int32),
                  gmin, gam)
    for _ in range(2):
        d3 = mask(d3, gam)
        gmin, gam = cands(d3)
        W, WI = merge(W, WI, gmin, gam)

    def check(W, WI, gmin, gam):
        t, ti = W[K - 1], WI[K - 1]
        bad = (gmin < t[None, :]) | ((gmin == t[None, :]) & (gam < ti[None, :]))
        return jnp.logical_not(jnp.any(bad))

    def cond(st):
        r, done = st[0], st[1]
        return jnp.logical_and(r < K, jnp.logical_not(done))

    def body(st):
        r, _, d3, gmin, gam, W, WI = st
        W, WI = merge(W, WI, gmin, gam)
        d3 = mask(d3, gam)
        gmin, gam = cands(d3)
        done = check(W, WI, gmin, gam)
        return r + 1, done, d3, gmin, gam, W, WI

    d3 = mask(d3, gam)
    gmin, gam = cands(d3)
    st = (jnp.int32(3), check(W, WI, gmin, gam), d3, gmin, gam, W, WI)
    _, _, _, _, _, W, WI = lax.while_loop(cond, body, st)
    idx_ref[0] = WI


def _topk(s_xyz, xyz):
    B, N, _ = s_xyz.shape
    S = xyz.shape[1]
    return pl.pallas_call(
        _topk_body,
        grid=(B, S // QT),
        in_specs=[
            pl.BlockSpec((1, QT, 3), lambda b, i: (b, i, 0)),
            pl.BlockSpec((1, N, 3), lambda b, i: (b, 0, 0)),
        ],
        out_specs=pl.BlockSpec((1, K, QT), lambda b, i: (b, 0, i)),
        out_shape=jax.ShapeDtypeStruct((B, K, S), jnp.int32),
    )(xyz, s_xyz)


def _make_sc_gather(BS):
    NC, NS = 2, 16
    NW = NC * NS
    QW = BS // NW
    NQ = 8
    NB = QW // NQ
    mesh = plsc.VectorSubcoreMesh(core_axis_name="c", subcore_axis_name="s")

    @functools.partial(
        pl.kernel, mesh=mesh,
        out_type=[
            jax.ShapeDtypeStruct((BS * K * OUTW,), jnp.float32),
            jax.ShapeDtypeStruct((BS * K * 3,), jnp.float32),
        ],
        scratch_types=[
            pltpu.VMEM((2, NQ * K), jnp.int32),
            pltpu.VMEM((2, NQ * K, ROWW), jnp.float32),
            pltpu.VMEM((NQ, 16), jnp.float32),
            pltpu.VMEM((NQ * K * OUTW + 16,), jnp.float32),
            pltpu.VMEM((NQ * K * 3 + 16,), jnp.float32),
            pltpu.SemaphoreType.DMA,
            pltpu.SemaphoreType.DMA,
        ],
    )
    def sc_gather(feat_hbm, gidx_hbm, qpad_hbm, newp_hbm, gxyz_hbm,
                  idx_v, rows_v, q_v, out_v, gx_v, sem0, sem1):
        wid = lax.axis_index("s") * NC + lax.axis_index("c")
        sems = (sem0, sem1)

        def start_gather(t, b):
            qbase = wid * QW + t * NQ
            pltpu.sync_copy(gidx_hbm.at[pl.ds(qbase * K, NQ * K)],
                            idx_v.at[b])
            pltpu.make_async_copy(feat_hbm.at[idx_v.at[b]], rows_v.at[b],
                                  sems[b]).start()

        start_gather(0, 0)

        def pair(tp, _):
            for b in range(2):
                t = tp * 2 + b
                @pl.when(t + 1 < NB)
                def _():
                    start_gather(t + 1, 1 - b)

                pltpu.make_async_copy(feat_hbm.at[idx_v.at[b]],
                                      rows_v.at[b], sems[b]).wait()
                qbase = wid * QW + t * NQ
                pltpu.sync_copy(qpad_hbm.at[pl.ds(qbase, NQ)], q_v)

                def body(i, _):
                    qvec = q_v[i, :]
                    for r in range(K):
                        row = i * K + r
                        d0 = row * OUTW
                        v0 = rows_v[b, row, pl.ds(0, 16)] - qvec
                        out_v[pl.ds(d0, 16)] = v0
                        for j in range(1, 5):
                            out_v[pl.ds(d0 + 16 * j, 16)] = (
                                rows_v[b, row, pl.ds(16 * j, 16)])
                        gx_v[pl.ds(row * 3, 16)] = v0
                    return 0

                lax.fori_loop(0, NQ, body, 0)
                pltpu.sync_copy(out_v.at[pl.ds(0, NQ * K * OUTW)],
                                newp_hbm.at[pl.ds(qbase * K * OUTW,
                                                  NQ * K * OUTW)])
                pltpu.sync_copy(gx_v.at[pl.ds(0, NQ * K * 3)],
                                gxyz_hbm.at[pl.ds(qbase * K * 3, NQ * K * 3)])
            return 0

        lax.fori_loop(0, NB // 2, pair, 0)

    return sc_gather


def kernel(s_xyz, xyz, s_points, nsample):
    B, N, _ = s_xyz.shape
    S = xyz.shape[1]
    D = s_points.shape[2]
    BS = B * S

    idx = _topk(s_xyz, xyz)
    idx = jnp.transpose(idx, (0, 2, 1))

    pad = jnp.zeros((B, N, ROWW - 3 - D), jnp.float32)
    feat = jnp.concatenate([s_xyz, s_points, pad], axis=-1).reshape(B * N, ROWW)
    gidx = (idx + (jnp.arange(B, dtype=jnp.int32) * N)[:, None, None]
            ).reshape(BS * K)
    qpad = jnp.concatenate(
        [xyz, jnp.zeros((B, S, 13), jnp.float32)], axis=-1).reshape(BS, 16)

    newp_rows, gxyz_flat = _make_sc_gather(BS)(feat, gidx, qpad)
    new_points = newp_rows.reshape(B, S, K, OUTW)
    grouped_xyz_norm = gxyz_flat.reshape(B, S, K, 3)
    return new_points, grouped_xyz_norm


# speedup vs baseline: 1.4728x; 1.0112x over previous
import functools

import jax
import jax.numpy as jnp
from jax import lax
from jax.experimental import pallas as pl
from jax.experimental.pallas import tpu as pltpu
from jax.experimental.pallas import tpu_sc as plsc

K = 16
QT = 256
ROWW = 128
OUTW = 3 + 64


G = 128


def _topk_body(xyz_ref, sxyz_ref, idx_ref):
    q = xyz_ref[0]
    s = sxyz_ref[0]
    n = s.shape[0]
    gs = n // G
    d = -2.0 * lax.dot_general(s, q, (((1,), (1,)), ((), ())),
                               preferred_element_type=jnp.float32)
    q2 = jnp.sum(q * q, axis=1)
    s2 = jnp.sum(s * s, axis=1)
    d = d + q2[None, :]
    d = d + s2[:, None]
    qt = d.shape[1]
    d3 = d.reshape(G, gs, qt)
    gbase = lax.broadcasted_iota(jnp.int32, (G, qt), 0) * gs
    inf = jnp.float32(jnp.inf)

    def cands(d3):
        v = d3
        i = None
        m = gs
        while m > 8:
            h = m // 2
            va, vb = v[:, :h], v[:, h:]
            take = vb < va
            if i is None:
                ia = lax.broadcasted_iota(jnp.int32, va.shape, 1)
                ib = ia + h
            else:
                ia, ib = i[:, :h], i[:, h:]
            v = jnp.where(take, vb, va)
            i = jnp.where(take, ib, ia)
            m = h
        gmin = jnp.min(v, axis=1)
        gaml = jnp.min(jnp.where(v == gmin[:, None, :], i, n), axis=1)
        return gmin, gaml, gaml + gbase

    def mask(d3, gaml):
        lio = lax.broadcasted_iota(jnp.int32, d3.shape, 1)
        return jnp.where(lio == gaml[:, None, :], inf, d3)

    def merge(W, WI, cv, ci):
        ev = jnp.concatenate([W, cv], axis=0)
        ei = jnp.concatenate([WI, ci], axis=0)
        nW, nWI = [], []
        for _ in range(K):
            w = jnp.min(ev, axis=0)
            wm = ev == w[None, :]
            wi = jnp.min(jnp.where(wm, ei, n), axis=0)
            nW.append(w)
            nWI.append(wi)
            ev = jnp.where(wm & (ei == wi[None, :]), inf, ev)
        return jnp.stack(nW), jnp.stack(nWI)

    gmin, gaml, gam = cands(d3)
    W, WI = merge(jnp.full((K, qt), inf), jnp.full((K, qt), n, jnp.int32),
                  gmin, gam)
    for _ in range(2):
        d3 = mask(d3, gaml)
        gmin, gaml, gam = cands(d3)
        W, WI = merge(W, WI, gmin, gam)

    def check(W, WI, gmin, gam):
        t, ti = W[K - 1], WI[K - 1]
        bad = (gmin < t[None, :]) | ((gmin == t[None, :]) & (gam < ti[None, :]))
        return jnp.logical_not(jnp.any(bad))

    def cond(st):
        r, done = st[0], st[1]
        return jnp.logical_and(r < K, jnp.logical_not(done))

    def body(st):
        r, _, d3, gmin, gaml, gam, W, WI = st
        W, WI = merge(W, WI, gmin, gam)
        d3 = mask(d3, gaml)
        gmin, gaml, gam = cands(d3)
        done = check(W, WI, gmin, gam)
        return r + 1, done, d3, gmin, gaml, gam, W, WI

    d3 = mask(d3, gaml)
    gmin, gaml, gam = cands(d3)
    st = (jnp.int32(3), check(W, WI, gmin, gam), d3, gmin, gaml, gam, W, WI)
    _, _, _, _, _, _, W, WI = lax.while_loop(cond, body, st)
    idx_ref[0] = WI


def _topk(s_xyz, xyz):
    B, N, _ = s_xyz.shape
    S = xyz.shape[1]
    return pl.pallas_call(
        _topk_body,
        grid=(B, S // QT),
        in_specs=[
            pl.BlockSpec((1, QT, 3), lambda b, i: (b, i, 0)),
            pl.BlockSpec((1, N, 3), lambda b, i: (b, 0, 0)),
        ],
        out_specs=pl.BlockSpec((1, K, QT), lambda b, i: (b, 0, i)),
        out_shape=jax.ShapeDtypeStruct((B, K, S), jnp.int32),
    )(xyz, s_xyz)


def _make_sc_gather(BS):
    NC, NS = 2, 16
    NW = NC * NS
    QW = BS // NW
    NQ = 8
    NB = QW // NQ
    mesh = plsc.VectorSubcoreMesh(core_axis_name="c", subcore_axis_name="s")

    @functools.partial(
        pl.kernel, mesh=mesh,
        out_type=[
            jax.ShapeDtypeStruct((BS * K * OUTW,), jnp.float32),
            jax.ShapeDtypeStruct((BS * K * 3,), jnp.float32),
        ],
        scratch_types=[
            pltpu.VMEM((2, NQ * K), jnp.int32),
            pltpu.VMEM((2, NQ * K, ROWW), jnp.float32),
            pltpu.VMEM((NQ, 16), jnp.float32),
            pltpu.VMEM((NQ * K * OUTW + 16,), jnp.float32),
            pltpu.VMEM((NQ * K * 3 + 16,), jnp.float32),
            pltpu.SemaphoreType.DMA,
            pltpu.SemaphoreType.DMA,
        ],
    )
    def sc_gather(feat_hbm, gidx_hbm, qpad_hbm, newp_hbm, gxyz_hbm,
                  idx_v, rows_v, q_v, out_v, gx_v, sem0, sem1):
        wid = lax.axis_index("s") * NC + lax.axis_index("c")
        sems = (sem0, sem1)

        def start_gather(t, b):
            qbase = wid * QW + t * NQ
            pltpu.sync_copy(gidx_hbm.at[pl.ds(qbase * K, NQ * K)],
                            idx_v.at[b])
            pltpu.make_async_copy(feat_hbm.at[idx_v.at[b]], rows_v.at[b],
                                  sems[b]).start()

        start_gather(0, 0)

        def pair(tp, _):
            for b in range(2):
                t = tp * 2 + b
                @pl.when(t + 1 < NB)
                def _():
                    start_gather(t + 1, 1 - b)

                pltpu.make_async_copy(feat_hbm.at[idx_v.at[b]],
                                      rows_v.at[b], sems[b]).wait()
                qbase = wid * QW + t * NQ
                pltpu.sync_copy(qpad_hbm.at[pl.ds(qbase, NQ)], q_v)

                def body(i, _):
                    qvec = q_v[i, :]
                    for r in range(K):
                        row = i * K + r
                        d0 = row * OUTW
                        v0 = rows_v[b, row, pl.ds(0, 16)] - qvec
                        out_v[pl.ds(d0, 16)] = v0
                        for j in range(1, 5):
                            out_v[pl.ds(d0 + 16 * j, 16)] = (
                                rows_v[b, row, pl.ds(16 * j, 16)])
                        gx_v[pl.ds(row * 3, 16)] = v0
                    return 0

                lax.fori_loop(0, NQ, body, 0)
                pltpu.sync_copy(out_v.at[pl.ds(0, NQ * K * OUTW)],
                                newp_hbm.at[pl.ds(qbase * K * OUTW,
                                                  NQ * K * OUTW)])
                pltpu.sync_copy(gx_v.at[pl.ds(0, NQ * K * 3)],
                                gxyz_hbm.at[pl.ds(qbase * K * 3, NQ * K * 3)])
            return 0

        lax.fori_loop(0, NB // 2, pair, 0)

    return sc_gather


def kernel(s_xyz, xyz, s_points, nsample):
    B, N, _ = s_xyz.shape
    S = xyz.shape[1]
    D = s_points.shape[2]
    BS = B * S

    idx = _topk(s_xyz, xyz)
    idx = jnp.transpose(idx, (0, 2, 1))

    pad = jnp.zeros((B, N, ROWW - 3 - D), jnp.float32)
    feat = jnp.concatenate([s_xyz, s_points, pad], axis=-1).reshape(B * N, ROWW)
    gidx = (idx + (jnp.arange(B, dtype=jnp.int32) * N)[:, None, None]
            ).reshape(BS * K)
    qpad = jnp.concatenate(
        [xyz, jnp.zeros((B, S, 13), jnp.float32)], axis=-1).reshape(BS, 16)

    newp_rows, gxyz_flat = _make_sc_gather(BS)(feat, gidx, qpad)
    new_points = newp_rows.reshape(B, S, K, OUTW)
    grouped_xyz_norm = gxyz_flat.reshape(B, S, K, 3)
    return new_points, grouped_xyz_norm
